# baseline (speedup 1.0000x reference)
import functools

import jax
import jax.numpy as jnp
from jax import lax
from jax.experimental import pallas as pl
from jax.experimental.pallas import tpu as pltpu
from jax.experimental.pallas import tpu_sc as plsc

_SEQ = 8192
_EMB = 768
_KEEP = 4096

_INT_MIN = -2147483648

_NC = 2
_NS = 16
_NW = _NC * _NS
_ROWS_PER_W = _KEEP // _NW

_ROW_BLK = 4096
_GRID = _SEQ // _ROW_BLK
_SL = _SEQ // 128


def _tc_scores_q_body(x_ref, w_ref, b_ref, q_ref, scores_s, v_s):
    i = pl.program_id(0)

    @pl.when(i == 0)
    def _():
        cls = x_ref[0:1, :]
        v = lax.dot_general(
            cls, w_ref[...], (((1,), (1,)), ((), ())),
            preferred_element_type=jnp.float32,
        )
        v_s[...] = v + b_ref[...].reshape(1, _EMB)

    for j in range(_ROW_BLK // 128):
        s = jnp.sum(x_ref[pl.ds(j * 128, 128), :] * v_s[...], axis=1)
        scores_s[pl.ds(i * (_ROW_BLK // 128) + j, 1), :] = s.reshape(1, 128)

    @pl.when(i == _GRID - 1)
    def _():
        sc = scores_s[...]
        u = lax.bitcast_convert_type(sc, jnp.int32)
        ki = jnp.where(u >= 0, u, ~u ^ _INT_MIN)
        r0 = lax.broadcasted_iota(jnp.int32, (_SL, 128), 0)
        c0 = lax.broadcasted_iota(jnp.int32, (_SL, 128), 1)
        is_cls = (r0 == 0) & (c0 == 0)
        ki = jnp.where(is_cls, _INT_MIN, ki)
        kb = ki ^ _INT_MIN

        p = jnp.zeros((1, 1), jnp.int32)
        v = jnp.zeros((1, 1), jnp.int32)
        r = jnp.full((1, 1), _KEEP, jnp.int32)
        for bit in range(31, -1, -1):
            bv = 1 << bit
            if bv >= 2**31:
                bv -= 2**32
            active = (kb & p) == v
            hi = active & ((kb & bv) != 0)
            c1 = jnp.sum(hi.astype(jnp.int32), keepdims=True)
            take = c1 >= r
            v = jnp.where(take, v | bv, v)
            r = jnp.where(take, r, r - c1)
            p = p | bv

        t = v ^ _INT_MIN
        gt = ki > t
        eq = ki == t
        cnt_gt = jnp.sum(gt.astype(jnp.int32), keepdims=True)
        extra = _KEEP - cnt_gt

        upper = (lax.broadcasted_iota(jnp.int32, (128, 128), 0)
                 <= lax.broadcasted_iota(jnp.int32, (128, 128), 1)
                 ).astype(jnp.float32)
        strict_low = (lax.broadcasted_iota(jnp.int32, (_SL, _SL), 1)
                      < lax.broadcasted_iota(jnp.int32, (_SL, _SL), 0)
                      ).astype(jnp.float32)

        def excl_cumsum(m):
            incl = jnp.dot(m, upper, preferred_element_type=jnp.float32)
            offs = jnp.dot(strict_low, jnp.sum(m, axis=1, keepdims=True),
                           preferred_element_type=jnp.float32)
            return incl + offs - m

        eq_rank = excl_cumsum(eq.astype(jnp.float32)).astype(jnp.int32)
        keep = gt | (eq & (eq_rank < extra))
        pos = excl_cumsum(keep.astype(jnp.float32)).astype(jnp.int32)
        q_ref[...] = 2 * pos + keep.astype(jnp.int32)


def _tc_scores_q(x, w, b2):
    q2d = pl.pallas_call(
        _tc_scores_q_body,
        grid=(_GRID,),
        in_specs=[
            pl.BlockSpec((_ROW_BLK, _EMB), lambda i: (i, 0)),
            pl.BlockSpec((_EMB, _EMB), lambda i: (0, 0)),
            pl.BlockSpec((_EMB,), lambda i: (0,)),
        ],
        out_specs=pl.BlockSpec((_SL, 128), lambda i: (0, 0)),
        out_shape=jax.ShapeDtypeStruct((_SL, 128), jnp.int32),
        scratch_shapes=[
            pltpu.VMEM((_SL, 128), jnp.float32),
            pltpu.VMEM((1, _EMB), jnp.float32),
        ],
    )(x, w, b2)
    return q2d.reshape(_SEQ)


_GCHUNK = 16


def _sc_body(x_hbm, q_hbm, out_hbm, q_v, idx_v, rows_v, cls_v, semg, semw):
    wid = lax.axis_index("s") * _NC + lax.axis_index("c")
    base = wid * _ROWS_PER_W
    pltpu.sync_copy(q_hbm, q_v)

    def lower_bound_vreg(t):
        p = jnp.int32(0)
        for sh in (256, 128, 64, 32, 16, 8, 4, 2, 1):
            cand = p + sh
            probe = q_v[pl.ds((cand - 1) * 16, 16)][0]
            p = jnp.where(probe < t, cand, p)
        last = q_v[pl.ds(p * 16, 16)][0]
        return p + (last < t).astype(jnp.int32)

    jlo = lower_bound_vreg(2 * base + 1)
    jhi = lower_bound_vreg(2 * (base + _ROWS_PER_W) + 1)
    jlo = jnp.maximum(jlo - 1, 0)

    def body(j, carry):
        v16 = q_v[pl.ds(j * 16, 16)]
        iv = j * 16 + lax.iota(jnp.int32, 16)
        slot = lax.shift_right_arithmetic(v16, 1)
        inr = ((v16 & 1) == 1) & (slot >= base) & (slot < base + _ROWS_PER_W)
        slot = jnp.where(inr, slot - base, 0)
        plsc.store_scatter(idx_v, [slot], iv, mask=inr)
        return carry

    lax.fori_loop(jlo, jhi, body, 0)

    nch = _ROWS_PER_W // _GCHUNK
    gathers = [
        pltpu.async_copy(
            x_hbm.at[idx_v.at[pl.ds(c * _GCHUNK, _GCHUNK)]],
            rows_v.at[pl.ds(c * _GCHUNK, _GCHUNK)], semg,
        )
        for c in range(nch)
    ]
    copies = []
    for c in range(nch):
        gathers[c].wait()
        copies.append(pltpu.async_copy(
            rows_v.at[pl.ds(c * _GCHUNK, _GCHUNK)],
            out_hbm.at[pl.ds(base + c * _GCHUNK, _GCHUNK)], semw,
        ))

    @pl.when(wid == 0)
    def _():
        pltpu.sync_copy(x_hbm.at[pl.ds(0, 1)], cls_v)
        pltpu.sync_copy(cls_v, out_hbm.at[pl.ds(_KEEP, 1)])

    for cp in copies:
        cp.wait()


@functools.cache
def _sc_select_gather():
    mesh = plsc.VectorSubcoreMesh(
        core_axis_name="c", subcore_axis_name="s",
        num_cores=_NC, num_subcores=_NS,
    )
    return pl.kernel(
        _sc_body,
        out_type=jax.ShapeDtypeStruct((_KEEP + 1, _EMB), jnp.float32),
        mesh=mesh,
        compiler_params=pltpu.CompilerParams(needs_layout_passes=False),
        scratch_types=[
            pltpu.VMEM((_SEQ,), jnp.int32),
            pltpu.VMEM((_ROWS_PER_W,), jnp.int32),
            pltpu.VMEM((_ROWS_PER_W, _EMB), jnp.float32),
            pltpu.VMEM((1, _EMB), jnp.float32),
            pltpu.SemaphoreType.DMA,
            pltpu.SemaphoreType.DMA,
        ],
    )


def kernel(cap_emb_norm, W, b):
    q = _tc_scores_q(cap_emb_norm, W, b)
    return _sc_select_gather()(cap_emb_norm, q)


# speedup vs baseline: 1.0052x; 1.0052x over previous
import functools

import jax
import jax.numpy as jnp
from jax import lax
from jax.experimental import pallas as pl
from jax.experimental.pallas import tpu as pltpu
from jax.experimental.pallas import tpu_sc as plsc

_SEQ = 8192
_EMB = 768
_KEEP = 4096

_INT_MIN = -2147483648

_NC = 2
_NS = 16
_NW = _NC * _NS
_ROWS_PER_W = _KEEP // _NW

_ROW_BLK = 2048
_GRID = _SEQ // _ROW_BLK
_SL = _SEQ // 128


def _tc_scores_q_body(x_ref, w_ref, b_ref, q_ref, scores_s, v_s):
    i = pl.program_id(0)

    @pl.when(i == 0)
    def _():
        cls = x_ref[0:1, :]
        v = lax.dot_general(
            cls, w_ref[...], (((1,), (1,)), ((), ())),
            preferred_element_type=jnp.float32,
        )
        v_s[...] = v + b_ref[...].reshape(1, _EMB)

    for j in range(_ROW_BLK // 128):
        s = jnp.sum(x_ref[pl.ds(j * 128, 128), :] * v_s[...], axis=1)
        scores_s[pl.ds(i * (_ROW_BLK // 128) + j, 1), :] = s.reshape(1, 128)

    @pl.when(i == _GRID - 1)
    def _():
        sc = scores_s[...]
        u = lax.bitcast_convert_type(sc, jnp.int32)
        ki = jnp.where(u >= 0, u, ~u ^ _INT_MIN)
        r0 = lax.broadcasted_iota(jnp.int32, (_SL, 128), 0)
        c0 = lax.broadcasted_iota(jnp.int32, (_SL, 128), 1)
        is_cls = (r0 == 0) & (c0 == 0)
        ki = jnp.where(is_cls, _INT_MIN, ki)
        kb = ki ^ _INT_MIN

        p = jnp.zeros((1, 1), jnp.int32)
        v = jnp.zeros((1, 1), jnp.int32)
        r = jnp.full((1, 1), _KEEP, jnp.int32)
        for bit in range(31, -1, -1):
            bv = 1 << bit
            if bv >= 2**31:
                bv -= 2**32
            active = (kb & p) == v
            hi = active & ((kb & bv) != 0)
            c1 = jnp.sum(hi.astype(jnp.int32), keepdims=True)
            take = c1 >= r
            v = jnp.where(take, v | bv, v)
            r = jnp.where(take, r, r - c1)
            p = p | bv

        t = v ^ _INT_MIN
        gt = ki > t
        eq = ki == t
        cnt_gt = jnp.sum(gt.astype(jnp.int32), keepdims=True)
        extra = _KEEP - cnt_gt

        upper = (lax.broadcasted_iota(jnp.int32, (128, 128), 0)
                 <= lax.broadcasted_iota(jnp.int32, (128, 128), 1)
                 ).astype(jnp.float32)
        strict_low = (lax.broadcasted_iota(jnp.int32, (_SL, _SL), 1)
                      < lax.broadcasted_iota(jnp.int32, (_SL, _SL), 0)
                      ).astype(jnp.float32)

        def excl_cumsum(m):
            incl = jnp.dot(m, upper, preferred_element_type=jnp.float32)
            offs = jnp.dot(strict_low, jnp.sum(m, axis=1, keepdims=True),
                           preferred_element_type=jnp.float32)
            return incl + offs - m

        eq_rank = excl_cumsum(eq.astype(jnp.float32)).astype(jnp.int32)
        keep = gt | (eq & (eq_rank < extra))
        pos = excl_cumsum(keep.astype(jnp.float32)).astype(jnp.int32)
        q_ref[...] = 2 * pos + keep.astype(jnp.int32)


def _tc_scores_q(x, w, b2):
    q2d = pl.pallas_call(
        _tc_scores_q_body,
        grid=(_GRID,),
        in_specs=[
            pl.BlockSpec((_ROW_BLK, _EMB), lambda i: (i, 0)),
            pl.BlockSpec((_EMB, _EMB), lambda i: (0, 0)),
            pl.BlockSpec((_EMB,), lambda i: (0,)),
        ],
        out_specs=pl.BlockSpec((_SL, 128), lambda i: (0, 0)),
        out_shape=jax.ShapeDtypeStruct((_SL, 128), jnp.int32),
        scratch_shapes=[
            pltpu.VMEM((_SL, 128), jnp.float32),
            pltpu.VMEM((1, _EMB), jnp.float32),
        ],
    )(x, w, b2)
    return q2d.reshape(_SEQ)


_GCHUNK = 16


def _sc_body(x_hbm, q_hbm, out_hbm, q_v, idx_v, rows_v, cls_v, semg, semw):
    wid = lax.axis_index("s") * _NC + lax.axis_index("c")
    base = wid * _ROWS_PER_W
    pltpu.sync_copy(q_hbm, q_v)

    def lower_bound_vreg(t):
        p = jnp.int32(0)
        for sh in (256, 128, 64, 32, 16, 8, 4, 2, 1):
            cand = p + sh
            probe = q_v[pl.ds((cand - 1) * 16, 16)][0]
            p = jnp.where(probe < t, cand, p)
        last = q_v[pl.ds(p * 16, 16)][0]
        return p + (last < t).astype(jnp.int32)

    jlo = lower_bound_vreg(2 * base + 1)
    jhi = lower_bound_vreg(2 * (base + _ROWS_PER_W) + 1)
    jlo = jnp.maximum(jlo - 1, 0)

    def body(j, carry):
        v16 = q_v[pl.ds(j * 16, 16)]
        iv = j * 16 + lax.iota(jnp.int32, 16)
        slot = lax.shift_right_arithmetic(v16, 1)
        inr = ((v16 & 1) == 1) & (slot >= base) & (slot < base + _ROWS_PER_W)
        slot = jnp.where(inr, slot - base, 0)
        plsc.store_scatter(idx_v, [slot], iv, mask=inr)
        return carry

    lax.fori_loop(jlo, jhi, body, 0)

    nch = _ROWS_PER_W // _GCHUNK
    gathers = [
        pltpu.async_copy(
            x_hbm.at[idx_v.at[pl.ds(c * _GCHUNK, _GCHUNK)]],
            rows_v.at[pl.ds(c * _GCHUNK, _GCHUNK)], semg,
        )
        for c in range(nch)
    ]
    copies = []
    for c in range(nch):
        gathers[c].wait()
        copies.append(pltpu.async_copy(
            rows_v.at[pl.ds(c * _GCHUNK, _GCHUNK)],
            out_hbm.at[pl.ds(base + c * _GCHUNK, _GCHUNK)], semw,
        ))

    @pl.when(wid == 0)
    def _():
        pltpu.sync_copy(x_hbm.at[pl.ds(0, 1)], cls_v)
        pltpu.sync_copy(cls_v, out_hbm.at[pl.ds(_KEEP, 1)])

    for cp in copies:
        cp.wait()


@functools.cache
def _sc_select_gather():
    mesh = plsc.VectorSubcoreMesh(
        core_axis_name="c", subcore_axis_name="s",
        num_cores=_NC, num_subcores=_NS,
    )
    return pl.kernel(
        _sc_body,
        out_type=jax.ShapeDtypeStruct((_KEEP + 1, _EMB), jnp.float32),
        mesh=mesh,
        compiler_params=pltpu.CompilerParams(
            needs_layout_passes=False, skip_device_barrier=True),
        scratch_types=[
            pltpu.VMEM((_SEQ,), jnp.int32),
            pltpu.VMEM((_ROWS_PER_W,), jnp.int32),
            pltpu.VMEM((_ROWS_PER_W, _EMB), jnp.float32),
            pltpu.VMEM((1, _EMB), jnp.float32),
            pltpu.SemaphoreType.DMA,
            pltpu.SemaphoreType.DMA,
        ],
    )


def kernel(cap_emb_norm, W, b):
    q = _tc_scores_q(cap_emb_norm, W, b)
    return _sc_select_gather()(cap_emb_norm, q)


# speedup vs baseline: 1.0558x; 1.0504x over previous
import functools

import jax
import jax.numpy as jnp
from jax import lax
from jax.experimental import pallas as pl
from jax.experimental.pallas import tpu as pltpu
from jax.experimental.pallas import tpu_sc as plsc

_SEQ = 8192
_EMB = 768
_KEEP = 4096

_INT_MIN = -2147483648

_NC = 2
_NS = 16
_NW = _NC * _NS
_ROWS_PER_W = _KEEP // _NW

_ROW_BLK = 2048
_GRID = _SEQ // _ROW_BLK
_SL = _SEQ // 128


def _tc_scores_q_body(x_ref, w_ref, b_ref, q_ref, scores_s, v_s):
    i = pl.program_id(0)

    @pl.when(i == 0)
    def _():
        cls = x_ref[0:1, :]
        v = lax.dot_general(
            cls, w_ref[...], (((1,), (1,)), ((), ())),
            preferred_element_type=jnp.float32,
        )
        v_s[...] = v + b_ref[...].reshape(1, _EMB)

    for j in range(_ROW_BLK // 128):
        s = jnp.sum(x_ref[pl.ds(j * 128, 128), :] * v_s[...], axis=1)
        scores_s[pl.ds(i * (_ROW_BLK // 128) + j, 1), :] = s.reshape(1, 128)

    @pl.when(i == _GRID - 1)
    def _():
        sc = scores_s[...]
        u = lax.bitcast_convert_type(sc, jnp.int32)
        ki = jnp.where(u >= 0, u, ~u ^ _INT_MIN)
        r0 = lax.broadcasted_iota(jnp.int32, (_SL, 128), 0)
        c0 = lax.broadcasted_iota(jnp.int32, (_SL, 128), 1)
        is_cls = (r0 == 0) & (c0 == 0)
        ki = jnp.where(is_cls, _INT_MIN, ki)
        kb = ki ^ _INT_MIN

        p = jnp.zeros((1, 1), jnp.int32)
        v = jnp.zeros((1, 1), jnp.int32)
        r = jnp.full((1, 1), _KEEP, jnp.int32)
        for bit in range(30, -1, -2):
            pv = 3 << bit
            if pv >= 2**31:
                pv -= 2**32
            active = (kb & p) == v
            nib = lax.shift_right_logical(kb, bit) & 3
            c3 = jnp.sum((active & (nib == 3)).astype(jnp.int32), keepdims=True)
            c2 = jnp.sum((active & (nib == 2)).astype(jnp.int32), keepdims=True)
            c1 = jnp.sum((active & (nib == 1)).astype(jnp.int32), keepdims=True)
            c32 = c3 + c2
            c321 = c32 + c1
            sel = jnp.where(
                r <= c3, 3, jnp.where(r <= c32, 2, jnp.where(r <= c321, 1, 0)))
            v = v | (sel << bit)
            r = r - jnp.where(
                r <= c3, 0, jnp.where(r <= c32, c3, jnp.where(r <= c321, c32, c321)))
            p = p | pv

        t = v ^ _INT_MIN
        gt = ki > t
        eq = ki == t
        cnt_gt = jnp.sum(gt.astype(jnp.int32), keepdims=True)
        extra = _KEEP - cnt_gt

        upper = (lax.broadcasted_iota(jnp.int32, (128, 128), 0)
                 <= lax.broadcasted_iota(jnp.int32, (128, 128), 1)
                 ).astype(jnp.float32)
        strict_low = (lax.broadcasted_iota(jnp.int32, (_SL, _SL), 1)
                      < lax.broadcasted_iota(jnp.int32, (_SL, _SL), 0)
                      ).astype(jnp.float32)

        def excl_cumsum(m):
            incl = jnp.dot(m, upper, preferred_element_type=jnp.float32)
            offs = jnp.dot(strict_low, jnp.sum(m, axis=1, keepdims=True),
                           preferred_element_type=jnp.float32)
            return incl + offs - m

        eq_rank = excl_cumsum(eq.astype(jnp.float32)).astype(jnp.int32)
        gt_rank = excl_cumsum(gt.astype(jnp.float32)).astype(jnp.int32)
        keep = gt | (eq & (eq_rank < extra))
        pos = gt_rank + jnp.minimum(eq_rank, extra)
        q_ref[...] = 2 * pos + keep.astype(jnp.int32)


def _tc_scores_q(x, w, b2):
    q2d = pl.pallas_call(
        _tc_scores_q_body,
        grid=(_GRID,),
        in_specs=[
            pl.BlockSpec((_ROW_BLK, _EMB), lambda i: (i, 0)),
            pl.BlockSpec((_EMB, _EMB), lambda i: (0, 0)),
            pl.BlockSpec((_EMB,), lambda i: (0,)),
        ],
        out_specs=pl.BlockSpec((_SL, 128), lambda i: (0, 0)),
        out_shape=jax.ShapeDtypeStruct((_SL, 128), jnp.int32),
        scratch_shapes=[
            pltpu.VMEM((_SL, 128), jnp.float32),
            pltpu.VMEM((1, _EMB), jnp.float32),
        ],
    )(x, w, b2)
    return q2d.reshape(_SEQ)


_GCHUNK = 16


def _sc_body(x_hbm, q_hbm, out_hbm, q_v, idx_v, rows_v, cls_v, semg, semw):
    wid = lax.axis_index("s") * _NC + lax.axis_index("c")
    base = wid * _ROWS_PER_W
    pltpu.sync_copy(q_hbm, q_v)

    def lower_bound_vreg(t):
        p = jnp.int32(0)
        for sh in (256, 128, 64, 32, 16, 8, 4, 2, 1):
            cand = p + sh
            probe = q_v[pl.ds((cand - 1) * 16, 16)][0]
            p = jnp.where(probe < t, cand, p)
        last = q_v[pl.ds(p * 16, 16)][0]
        return p + (last < t).astype(jnp.int32)

    jlo = lower_bound_vreg(2 * base + 1)
    jhi = lower_bound_vreg(2 * (base + _ROWS_PER_W) + 1)
    jlo = jnp.maximum(jlo - 1, 0)

    def body(j, carry):
        v16 = q_v[pl.ds(j * 16, 16)]
        iv = j * 16 + lax.iota(jnp.int32, 16)
        slot = lax.shift_right_arithmetic(v16, 1)
        inr = ((v16 & 1) == 1) & (slot >= base) & (slot < base + _ROWS_PER_W)
        slot = jnp.where(inr, slot - base, 0)
        plsc.store_scatter(idx_v, [slot], iv, mask=inr)
        return carry

    lax.fori_loop(jlo, jhi, body, 0)

    nch = _ROWS_PER_W // _GCHUNK
    gathers = [
        pltpu.async_copy(
            x_hbm.at[idx_v.at[pl.ds(c * _GCHUNK, _GCHUNK)]],
            rows_v.at[pl.ds(c * _GCHUNK, _GCHUNK)], semg,
        )
        for c in range(nch)
    ]
    copies = []
    for c in range(nch):
        gathers[c].wait()
        copies.append(pltpu.async_copy(
            rows_v.at[pl.ds(c * _GCHUNK, _GCHUNK)],
            out_hbm.at[pl.ds(base + c * _GCHUNK, _GCHUNK)], semw,
        ))

    @pl.when(wid == 0)
    def _():
        pltpu.sync_copy(x_hbm.at[pl.ds(0, 1)], cls_v)
        pltpu.sync_copy(cls_v, out_hbm.at[pl.ds(_KEEP, 1)])

    for cp in copies:
        cp.wait()


@functools.cache
def _sc_select_gather():
    mesh = plsc.VectorSubcoreMesh(
        core_axis_name="c", subcore_axis_name="s",
        num_cores=_NC, num_subcores=_NS,
    )
    return pl.kernel(
        _sc_body,
        out_type=jax.ShapeDtypeStruct((_KEEP + 1, _EMB), jnp.float32),
        mesh=mesh,
        compiler_params=pltpu.CompilerParams(needs_layout_passes=False),
        scratch_types=[
            pltpu.VMEM((_SEQ,), jnp.int32),
            pltpu.VMEM((_ROWS_PER_W,), jnp.int32),
            pltpu.VMEM((_ROWS_PER_W, _EMB), jnp.float32),
            pltpu.VMEM((1, _EMB), jnp.float32),
            pltpu.SemaphoreType.DMA,
            pltpu.SemaphoreType.DMA,
        ],
    )


def kernel(cap_emb_norm, W, b):
    q = _tc_scores_q(cap_emb_norm, W, b)
    return _sc_select_gather()(cap_emb_norm, q)
